# Initial kernel scaffold; baseline (speedup 1.0000x reference)
import jax
import jax.numpy as jnp
from jax.experimental import pallas as pl


def kernel(x, edge_index, batch, W_exp, b_exp, Wc, bc):
    raise NotImplementedError("write your pallas kernel here")


# speedup vs baseline: 8.9946x; 8.9946x over previous
import functools

import jax
import jax.numpy as jnp
from jax import lax
from jax.experimental import pallas as pl
from jax.experimental.pallas import tpu as pltpu
from jax.experimental.pallas import tpu_sc as plsc

N = 10000
E = 320000
D = 128
G = 256
L = 5

NC = 2
NS = 16
CHUNK = 128
NCH = E // CHUNK
CH_PER_CORE = NCH // NC
RPS = 624
TAIL = N - NS * RPS
TAIL_OFF = NS * RPS
ZR = 208

BLK = 2000
_PREC = lax.Precision.HIGHEST


def _dotT(a, w):
    return lax.dot_general(a, w, (((1,), (1,)), ((), ())), precision=_PREC)


def _sc_mesh():
    return plsc.VectorSubcoreMesh(core_axis_name="c", subcore_axis_name="s")


def _deg_sc(dst2d):
    @functools.partial(
        pl.kernel,
        out_type=jax.ShapeDtypeStruct((NC, N, 16), jnp.float32),
        mesh=_sc_mesh(),
        scratch_types=[
            pltpu.VMEM((CHUNK,), jnp.int32),
            pltpu.VMEM((CHUNK, 16), jnp.float32),
            pltpu.VMEM((ZR, 16), jnp.float32),
            pltpu.VMEM_SHARED((N, 16), jnp.float32),
        ],
    )
    def k(dst_hbm, out_hbm, dst_v, ones_v, zero_v, acc_sh):
        c = lax.axis_index("c")
        s = lax.axis_index("s")

        @pl.loop(0, CHUNK)
        def _(i):
            ones_v[i, :] = jnp.ones((16,), jnp.float32)

        @pl.loop(0, ZR)
        def _(i):
            zero_v[i, :] = jnp.zeros((16,), jnp.float32)

        base = s * RPS
        for t in range(RPS // ZR):
            pltpu.sync_copy(zero_v, acc_sh.at[pl.ds(base + t * ZR, ZR)])

        @pl.when(s == 0)
        def _():
            pltpu.sync_copy(zero_v.at[pl.ds(0, TAIL)],
                            acc_sh.at[pl.ds(TAIL_OFF, TAIL)])

        plsc.subcore_barrier()

        @pl.loop(c * CH_PER_CORE + s, (c + 1) * CH_PER_CORE, step=NS)
        def _(ch):
            pltpu.sync_copy(dst_hbm.at[ch], dst_v)
            pltpu.sync_copy(ones_v, acc_sh.at[dst_v], add=True)

        plsc.subcore_barrier()
        pltpu.sync_copy(acc_sh.at[pl.ds(base, RPS)],
                        out_hbm.at[c, pl.ds(base, RPS)])

        @pl.when(s == 0)
        def _():
            pltpu.sync_copy(acc_sh.at[pl.ds(TAIL_OFF, TAIL)],
                            out_hbm.at[c, pl.ds(TAIL_OFF, TAIL)])

    return k(dst2d)


def _edge_sc(hp, src2d, dst2d):
    @functools.partial(
        pl.kernel,
        out_type=jax.ShapeDtypeStruct((NC, N, D), jnp.float32),
        mesh=_sc_mesh(),
        scratch_types=[
            pltpu.VMEM((CHUNK,), jnp.int32),
            pltpu.VMEM((CHUNK,), jnp.int32),
            pltpu.VMEM((CHUNK, D), jnp.float32),
            pltpu.VMEM((ZR, D), jnp.float32),
            pltpu.VMEM_SHARED((N, D), jnp.float32),
            pltpu.SemaphoreType.DMA,
        ],
    )
    def k(hp_hbm, src_hbm, dst_hbm, out_hbm,
          src_v, dst_v, rows_v, zero_v, acc_sh, sem):
        c = lax.axis_index("c")
        s = lax.axis_index("s")

        @pl.loop(0, ZR)
        def _(i):
            @pl.loop(0, D, step=16)
            def _(j):
                zero_v[i, pl.ds(j, 16)] = jnp.zeros((16,), jnp.float32)

        base = s * RPS
        for t in range(RPS // ZR):
            pltpu.sync_copy(zero_v, acc_sh.at[pl.ds(base + t * ZR, ZR)])

        @pl.when(s == 0)
        def _():
            pltpu.sync_copy(zero_v.at[pl.ds(0, TAIL)],
                            acc_sh.at[pl.ds(TAIL_OFF, TAIL)])

        plsc.subcore_barrier()

        @pl.loop(c * CH_PER_CORE + s, (c + 1) * CH_PER_CORE, step=NS)
        def _(ch):
            pltpu.sync_copy(src_hbm.at[ch], src_v)
            pltpu.sync_copy(dst_hbm.at[ch], dst_v)
            pltpu.async_copy(hp_hbm.at[src_v], rows_v, sem).wait()
            pltpu.sync_copy(rows_v, acc_sh.at[dst_v], add=True)

        plsc.subcore_barrier()
        pltpu.sync_copy(acc_sh.at[pl.ds(base, RPS)],
                        out_hbm.at[c, pl.ds(base, RPS)])

        @pl.when(s == 0)
        def _():
            pltpu.sync_copy(acc_sh.at[pl.ds(TAIL_OFF, TAIL)],
                            out_hbm.at[c, pl.ds(TAIL_OFF, TAIL)])

    return k(hp, src2d, dst2d)


def _prep_tc(xp, wep, be2, w0, dega, degb):
    def body(xp_ref, we_ref, be_ref, w0_ref, dga_ref, dgb_ref,
             atoms_ref, hp_ref, dis_ref):
        a = jnp.log(xp_ref[...] + 1.0)
        a = _dotT(a, we_ref[...]) + be_ref[...]
        deg = dga_ref[:, 0:1] + dgb_ref[:, 0:1] + 2.0
        dis = jnp.broadcast_to(lax.rsqrt(deg), (BLK, D))
        atoms_ref[...] = a
        hp_ref[...] = _dotT(a, w0_ref[...]) * dis
        dis_ref[...] = dis

    fdd = jax.ShapeDtypeStruct((N, D), jnp.float32)
    return pl.pallas_call(
        body,
        grid=(N // BLK,),
        in_specs=[
            pl.BlockSpec((BLK, 16), lambda i: (i, 0)),
            pl.BlockSpec((D, 16), lambda i: (0, 0)),
            pl.BlockSpec((1, D), lambda i: (0, 0)),
            pl.BlockSpec((D, D), lambda i: (0, 0)),
            pl.BlockSpec((BLK, 16), lambda i: (i, 0)),
            pl.BlockSpec((BLK, 16), lambda i: (i, 0)),
        ],
        out_specs=[
            pl.BlockSpec((BLK, D), lambda i: (i, 0)),
            pl.BlockSpec((BLK, D), lambda i: (i, 0)),
            pl.BlockSpec((BLK, D), lambda i: (i, 0)),
        ],
        out_shape=[fdd, fdd, fdd],
    )(xp, wep, be2, w0, dega, degb)


def _layer_tc(atoms, hp, acca, accb, dis, b2, wnext):
    def body(at_ref, hp_ref, aa_ref, ab_ref, dis_ref, b_ref, wn_ref,
             ao_ref, ho_ref):
        dis = dis_ref[...]
        out = dis * (aa_ref[...] + ab_ref[...] + 2.0 * hp_ref[...]) + b_ref[...]
        a = at_ref[...] + jnp.maximum(out, 0.0)
        ao_ref[...] = a
        ho_ref[...] = _dotT(a, wn_ref[...]) * dis

    fdd = jax.ShapeDtypeStruct((N, D), jnp.float32)
    return pl.pallas_call(
        body,
        grid=(N // BLK,),
        in_specs=[
            pl.BlockSpec((BLK, D), lambda i: (i, 0)),
            pl.BlockSpec((BLK, D), lambda i: (i, 0)),
            pl.BlockSpec((BLK, D), lambda i: (i, 0)),
            pl.BlockSpec((BLK, D), lambda i: (i, 0)),
            pl.BlockSpec((BLK, D), lambda i: (i, 0)),
            pl.BlockSpec((1, D), lambda i: (0, 0)),
            pl.BlockSpec((D, D), lambda i: (0, 0)),
        ],
        out_specs=[
            pl.BlockSpec((BLK, D), lambda i: (i, 0)),
            pl.BlockSpec((BLK, D), lambda i: (i, 0)),
        ],
        out_shape=[fdd, fdd],
    )(atoms, hp, acca, accb, dis, b2, wnext)


def _final_tc(atoms, hp, acca, accb, dis, b2, batch3d):
    def body(at_ref, hp_ref, aa_ref, ab_ref, dis_ref, b_ref, bt_ref,
             pool_ref):
        dis = dis_ref[...]
        out = dis * (aa_ref[...] + ab_ref[...] + 2.0 * hp_ref[...]) + b_ref[...]
        a = at_ref[...] + jnp.maximum(out, 0.0)
        bvec = bt_ref[0, 0, :]
        oh = (lax.broadcasted_iota(jnp.int32, (G, BLK), 0)
              == bvec[None, :]).astype(jnp.float32)
        p = jnp.dot(oh, a, precision=_PREC)

        @pl.when(pl.program_id(0) == 0)
        def _():
            pool_ref[...] = p

        @pl.when(pl.program_id(0) > 0)
        def _():
            pool_ref[...] += p

    return pl.pallas_call(
        body,
        grid=(N // BLK,),
        in_specs=[
            pl.BlockSpec((BLK, D), lambda i: (i, 0)),
            pl.BlockSpec((BLK, D), lambda i: (i, 0)),
            pl.BlockSpec((BLK, D), lambda i: (i, 0)),
            pl.BlockSpec((BLK, D), lambda i: (i, 0)),
            pl.BlockSpec((BLK, D), lambda i: (i, 0)),
            pl.BlockSpec((1, D), lambda i: (0, 0)),
            pl.BlockSpec((1, 1, BLK), lambda i: (i, 0, 0)),
        ],
        out_specs=pl.BlockSpec((G, D), lambda i: (0, 0)),
        out_shape=jax.ShapeDtypeStruct((G, D), jnp.float32),
    )(atoms, hp, acca, accb, dis, b2, batch3d)


def kernel(x, edge_index, batch, W_exp, b_exp, Wc, bc):
    src2d = edge_index[0].astype(jnp.int32).reshape(NCH, CHUNK)
    dst2d = edge_index[1].astype(jnp.int32).reshape(NCH, CHUNK)
    xp = jnp.pad(x.astype(jnp.float32), ((0, 0), (0, 5)))
    wep = jnp.pad(W_exp, ((0, 0), (0, 5)))
    batch3d = batch.astype(jnp.int32).reshape(N // BLK, 1, BLK)

    deg = _deg_sc(dst2d)
    atoms, hp, dis = _prep_tc(xp, wep, b_exp.reshape(1, D), Wc[0],
                              deg[0], deg[1])
    for i in range(L):
        acc = _edge_sc(hp, src2d, dst2d)
        if i < L - 1:
            atoms, hp = _layer_tc(atoms, hp, acc[0], acc[1], dis,
                                  bc[i].reshape(1, D), Wc[i + 1])
        else:
            pool = _final_tc(atoms, hp, acc[0], acc[1], dis,
                             bc[i].reshape(1, D), batch3d)
    return pool
